# baseline (speedup 1.0000x reference)
import jax
import jax.numpy as jnp
from jax.experimental import pallas as pl
from jax.experimental.pallas import tpu as pltpu

B = 4096
IN_DIM = 512
OUT_DIM = 8192
N_NEURON = 256
N_LEAF = 4
LEAF_DIM = 8

ROW_BLOCK = 256


def _build_m_kernel(idx_ref, w_ref, m_ref):
    row = jax.lax.broadcasted_iota(jnp.int32, (IN_DIM, OUT_DIM), 0)
    onehot = jnp.where(row == idx_ref[0][None, :], w_ref[0][None, :], 0.0)
    m_ref[...] = onehot.astype(jnp.bfloat16)


def _gather_mm_kernel(x_ref, m_ref, out_ref):
    hi = x_ref[...].astype(jnp.bfloat16)
    sel = jnp.dot(hi, m_ref[...], preferred_element_type=jnp.float32)
    out_ref[...] = sel.reshape(out_ref.shape)


def kernel(x, idx, w):
    ld = N_LEAF * LEAF_DIM
    idx2 = idx.reshape(N_NEURON, ld).T.reshape(1, OUT_DIM)
    w2 = w.reshape(N_NEURON, ld).T.reshape(1, OUT_DIM)

    m = pl.pallas_call(
        _build_m_kernel,
        out_shape=jax.ShapeDtypeStruct((IN_DIM, OUT_DIM), jnp.bfloat16),
    )(idx2, w2)

    phys = pl.pallas_call(
        _gather_mm_kernel,
        grid=(B // ROW_BLOCK,),
        in_specs=[
            pl.BlockSpec((ROW_BLOCK, IN_DIM), lambda i: (i, 0)),
            pl.BlockSpec((IN_DIM, OUT_DIM), lambda i: (0, 0)),
        ],
        out_specs=pl.BlockSpec((ROW_BLOCK, ld, N_NEURON), lambda i: (i, 0, 0)),
        out_shape=jax.ShapeDtypeStruct((B, ld, N_NEURON), x.dtype),
        compiler_params=pltpu.CompilerParams(
            dimension_semantics=("parallel",),
        ),
    )(x, m)
    out = phys.reshape(B, N_LEAF, LEAF_DIM, N_NEURON).transpose(0, 3, 1, 2)
    return out


# speedup vs baseline: 1.0617x; 1.0617x over previous
import jax
import jax.numpy as jnp
from jax.experimental import pallas as pl
from jax.experimental.pallas import tpu as pltpu

B = 4096
IN_DIM = 512
OUT_DIM = 8192
N_NEURON = 256
N_LEAF = 4
LEAF_DIM = 8

ROW_BLOCK = 256
N_CHUNK = 4
CHUNK = OUT_DIM // N_CHUNK
CHUNK_LD = CHUNK // N_NEURON


def _gather_mm_kernel(x_ref, idx_ref, w_ref, out_ref, m_ref):
    @pl.when(pl.program_id(0) == 0)
    def _build_selection_matrix():
        row = jax.lax.broadcasted_iota(jnp.int32, (IN_DIM, OUT_DIM), 0)
        onehot = jnp.where(row == idx_ref[0][None, :], w_ref[0][None, :], 0.0)
        m_ref[...] = onehot.astype(jnp.bfloat16)

    hi = x_ref[...].astype(jnp.bfloat16)
    for j in range(OUT_DIM // N_NEURON):
        sel = jnp.dot(hi, m_ref[:, j * N_NEURON:(j + 1) * N_NEURON],
                      preferred_element_type=jnp.float32)
        out_ref[:, j, :] = sel


def kernel(x, idx, w):
    ld = N_LEAF * LEAF_DIM
    idx2 = idx.reshape(N_NEURON, ld).T.reshape(1, OUT_DIM)
    w2 = w.reshape(N_NEURON, ld).T.reshape(1, OUT_DIM)
    phys = pl.pallas_call(
        _gather_mm_kernel,
        grid=(B // ROW_BLOCK,),
        in_specs=[
            pl.BlockSpec((ROW_BLOCK, IN_DIM), lambda i: (i, 0)),
            pl.BlockSpec((1, OUT_DIM), lambda i: (0, 0)),
            pl.BlockSpec((1, OUT_DIM), lambda i: (0, 0)),
        ],
        out_specs=pl.BlockSpec((ROW_BLOCK, ld, N_NEURON), lambda i: (i, 0, 0)),
        out_shape=jax.ShapeDtypeStruct((B, ld, N_NEURON), x.dtype),
        scratch_shapes=[pltpu.VMEM((IN_DIM, OUT_DIM), jnp.bfloat16)],
    )(x, idx2, w2)
    out = phys.reshape(B, N_LEAF, LEAF_DIM, N_NEURON).transpose(0, 3, 1, 2)
    return out


# speedup vs baseline: 1.0803x; 1.0175x over previous
import jax
import jax.numpy as jnp
from jax.experimental import pallas as pl
from jax.experimental.pallas import tpu as pltpu

B = 4096
IN_DIM = 512
OUT_DIM = 8192
N_NEURON = 256
N_LEAF = 4
LEAF_DIM = 8

ROW_BLOCK = 256
N_CHUNK = 4
CHUNK = OUT_DIM // N_CHUNK
CHUNK_LD = CHUNK // N_NEURON


def _gather_mm_kernel(x_ref, idx_ref, w_ref, out_ref, m_ref):
    @pl.when(pl.program_id(0) == 0)
    def _build_selection_matrix():
        row = jax.lax.broadcasted_iota(jnp.int32, (IN_DIM, OUT_DIM), 0)
        onehot = jnp.where(row == idx_ref[0][None, :], w_ref[0][None, :], 0.0)
        m_ref[...] = onehot.astype(jnp.bfloat16)

    hi = x_ref[...].astype(jnp.bfloat16)
    ld = N_LEAF * LEAF_DIM
    for g in range(ld // 8):
        sel = jnp.dot(hi, m_ref[:, g * 8 * N_NEURON:(g + 1) * 8 * N_NEURON],
                      preferred_element_type=jnp.float32)
        out_ref[:, g * 8:(g + 1) * 8, :] = sel.reshape(ROW_BLOCK, 8, N_NEURON)


def kernel(x, idx, w):
    ld = N_LEAF * LEAF_DIM
    idx2 = idx.reshape(N_NEURON, ld).T.reshape(1, OUT_DIM)
    w2 = w.reshape(N_NEURON, ld).T.reshape(1, OUT_DIM)
    phys = pl.pallas_call(
        _gather_mm_kernel,
        grid=(B // ROW_BLOCK,),
        in_specs=[
            pl.BlockSpec((ROW_BLOCK, IN_DIM), lambda i: (i, 0)),
            pl.BlockSpec((1, OUT_DIM), lambda i: (0, 0)),
            pl.BlockSpec((1, OUT_DIM), lambda i: (0, 0)),
        ],
        out_specs=pl.BlockSpec((ROW_BLOCK, ld, N_NEURON), lambda i: (i, 0, 0)),
        out_shape=jax.ShapeDtypeStruct((B, ld, N_NEURON), x.dtype),
        scratch_shapes=[pltpu.VMEM((IN_DIM, OUT_DIM), jnp.bfloat16)],
    )(x, idx2, w2)
    out = phys.reshape(B, N_LEAF, LEAF_DIM, N_NEURON).transpose(0, 3, 1, 2)
    return out


# speedup vs baseline: 1.1946x; 1.1059x over previous
import jax
import jax.numpy as jnp
from jax.experimental import pallas as pl
from jax.experimental.pallas import tpu as pltpu

B = 4096
IN_DIM = 512
OUT_DIM = 8192
N_NEURON = 256
N_LEAF = 4
LEAF_DIM = 8

ROW_BLOCK = 256
N_CHUNK = 4
CHUNK = OUT_DIM // N_CHUNK
CHUNK_LD = CHUNK // N_NEURON


def _gather_mm_kernel(x_ref, idx_ref, w_ref, out_ref, m_ref):
    @pl.when(pl.program_id(0) == 0)
    def _build_selection_matrix():
        row = jax.lax.broadcasted_iota(jnp.int32, (IN_DIM, OUT_DIM), 0)
        onehot = jnp.where(row == idx_ref[0][None, :], w_ref[0][None, :], 0.0)
        m_ref[...] = onehot.astype(jnp.bfloat16)

    hi = x_ref[...].astype(jnp.bfloat16)
    ld = N_LEAF * LEAF_DIM
    for g in range(ld // 8):
        sel = jnp.dot(hi, m_ref[:, g * 8 * N_NEURON:(g + 1) * 8 * N_NEURON],
                      preferred_element_type=jnp.float32)
        t = sel.astype(jnp.bfloat16).reshape(ROW_BLOCK, 8, N_NEURON)
        out_ref[:, g * 8:(g + 1) * 8, :] = t.astype(jnp.float32)


def kernel(x, idx, w):
    ld = N_LEAF * LEAF_DIM
    idx2 = idx.reshape(N_NEURON, ld).T.reshape(1, OUT_DIM)
    w2 = w.reshape(N_NEURON, ld).T.reshape(1, OUT_DIM)
    phys = pl.pallas_call(
        _gather_mm_kernel,
        grid=(B // ROW_BLOCK,),
        in_specs=[
            pl.BlockSpec((ROW_BLOCK, IN_DIM), lambda i: (i, 0)),
            pl.BlockSpec((1, OUT_DIM), lambda i: (0, 0)),
            pl.BlockSpec((1, OUT_DIM), lambda i: (0, 0)),
        ],
        out_specs=pl.BlockSpec((ROW_BLOCK, ld, N_NEURON), lambda i: (i, 0, 0)),
        out_shape=jax.ShapeDtypeStruct((B, ld, N_NEURON), x.dtype),
        scratch_shapes=[pltpu.VMEM((IN_DIM, OUT_DIM), jnp.bfloat16)],
    )(x, idx2, w2)
    out = phys.reshape(B, N_LEAF, LEAF_DIM, N_NEURON).transpose(0, 3, 1, 2)
    return out
